# baseline (speedup 1.0000x reference)
import functools

import jax
import jax.numpy as jnp
from jax import lax
from jax.experimental import pallas as pl
from jax.experimental.pallas import tpu as pltpu
from jax.experimental.pallas import tpu_sc as plsc

N = 10000
C = 128
E = 320000
NC = 2
NS = 16
NW = NC * NS
K = 64
CH = 160
CB = 40
NB = CH // CB
EPT = CH * K
E_PAD = NW * EPT
N_PAD = 10240
RPT = N_PAD // NS
LANES = 16


def _sc_scatter_sum(xp, src3, dst3):
    mesh = plsc.VectorSubcoreMesh(core_axis_name="c", subcore_axis_name="s")

    @functools.partial(
        pl.kernel,
        out_type=jax.ShapeDtypeStruct((NC, N_PAD, C), jnp.float32),
        mesh=mesh,
        scratch_types=[
            pltpu.VMEM((CB, K), jnp.int32),
            pltpu.VMEM((CB, K), jnp.int32),
            [pltpu.VMEM((K, C), jnp.float32)] * 4,
            pltpu.VMEM_SHARED((N_PAD, C), jnp.float32),
            [pltpu.SemaphoreType.DMA] * 4,
            [pltpu.SemaphoreType.DMA] * 4,
        ],
    )
    def body(x_hbm, src_hbm, dst_hbm, out_hbm, src_v, dst_v, msgs_v, acc,
             gsem, ssem):
        core = lax.axis_index("c")
        sid = lax.axis_index("s")
        wid = core * NS + sid

        def zrow(i, carry):
            for j in range(C // LANES):
                msgs_v[0][i, pl.ds(j * LANES, LANES)] = jnp.zeros(
                    (LANES,), jnp.float32)
            return carry
        lax.fori_loop(0, K, zrow, 0)
        for t in range(RPT // K):
            pltpu.sync_copy(msgs_v[0], acc.at[pl.ds(sid * RPT + t * K, K)])

        plsc.subcore_barrier()

        def gather(j, b):
            return pltpu.make_async_copy(x_hbm.at[src_v.at[j]],
                                         msgs_v[b], gsem[b])

        def scatter(j, b):
            return pltpu.make_async_copy(msgs_v[b],
                                         acc.at[dst_v.at[j]], ssem[b])

        for blk in range(NB):
            pltpu.sync_copy(src_hbm.at[wid, pl.ds(blk * CB, CB)], src_v)
            pltpu.sync_copy(dst_hbm.at[wid, pl.ds(blk * CB, CB)], dst_v)
            gather(0, 0).start()
            gather(1, 1).start()

            def quad(p, carry):
                for b in range(4):
                    j = 4 * p + b
                    b2 = (b + 2) % 4
                    gather(j, b).wait()
                    scatter(j, b).start(add=True)

                    @pl.when((j >= 2) & (j + 2 < CB))
                    def _():
                        scatter(j - 2, b2).wait()

                    @pl.when(j + 2 < CB)
                    def _():
                        gather(j + 2, b2).start()
                return carry
            lax.fori_loop(0, CB // 4, quad, 0)
            for t in range(4):
                scatter(CB - 4 + t, (CB - 4 + t) % 4).wait()

        plsc.subcore_barrier()
        pltpu.sync_copy(acc.at[pl.ds(sid * RPT, RPT)],
                        out_hbm.at[core, pl.ds(sid * RPT, RPT)])

    return body(xp, src3, dst3)


def _combine_body(p_ref, o_ref):
    s = p_ref[0, :N, :] + p_ref[1, :N, :]
    o_ref[...] = s.T


def _tc_combine(partial):
    return pl.pallas_call(
        _combine_body,
        in_specs=[pl.BlockSpec((NC, N_PAD, C), lambda: (0, 0, 0))],
        out_specs=pl.BlockSpec((C, N), lambda: (0, 0)),
        out_shape=jax.ShapeDtypeStruct((C, N), jnp.float32),
    )(partial)


def kernel(x, edge_index):
    xp = jnp.transpose(x.reshape(C, N))
    src = edge_index[0].astype(jnp.int32)
    dst = edge_index[1].astype(jnp.int32)
    ppt = EPT - E // NW
    w = jnp.arange(NW, dtype=jnp.int32)[:, None]
    i = jnp.arange(ppt, dtype=jnp.int32)[None, :]
    pad_src = (w * 331 + i * 41) % N
    pad_dst = N + (i + w * 15) % (N_PAD - N)
    src3 = jnp.concatenate(
        [src.reshape(NW, E // NW), pad_src], axis=1).reshape(NW, CH, K)
    dst3 = jnp.concatenate(
        [dst.reshape(NW, E // NW), pad_dst], axis=1).reshape(NW, CH, K)
    partial = _sc_scatter_sum(xp, src3, dst3)
    out_cn = _tc_combine(partial)
    return out_cn.reshape(1, C, N, 1)


# speedup vs baseline: 1.0379x; 1.0379x over previous
import functools

import jax
import jax.numpy as jnp
from jax import lax
from jax.experimental import pallas as pl
from jax.experimental.pallas import tpu as pltpu
from jax.experimental.pallas import tpu_sc as plsc

N = 10000
C = 128
E = 320000
NC = 2
NS = 16
NW = NC * NS
K = 125
CH = 80
CB = 40
NB = CH // CB
EPT = CH * K
N_PAD = 10240
RPT = N_PAD // NS
LANES = 16


def _sc_scatter_sum(xp, src3, dst3):
    mesh = plsc.VectorSubcoreMesh(core_axis_name="c", subcore_axis_name="s")

    @functools.partial(
        pl.kernel,
        out_type=jax.ShapeDtypeStruct((NC, N_PAD, C), jnp.float32),
        mesh=mesh,
        scratch_types=[
            pltpu.VMEM((CB, K), jnp.int32),
            pltpu.VMEM((CB, K), jnp.int32),
            [pltpu.VMEM((K, C), jnp.float32)] * 2,
            pltpu.VMEM_SHARED((N_PAD, C), jnp.float32),
            [pltpu.SemaphoreType.DMA] * 2,
            [pltpu.SemaphoreType.DMA] * 2,
        ],
    )
    def body(x_hbm, src_hbm, dst_hbm, out_hbm, src_v, dst_v, msgs_v, acc,
             gsem, ssem):
        core = lax.axis_index("c")
        sid = lax.axis_index("s")
        wid = core * NS + sid

        def zrow(i, carry):
            for j in range(C // LANES):
                msgs_v[0][i, pl.ds(j * LANES, LANES)] = jnp.zeros(
                    (LANES,), jnp.float32)
            return carry
        lax.fori_loop(0, K, zrow, 0)
        zc = 40
        for t in range(RPT // zc):
            pltpu.sync_copy(msgs_v[0].at[pl.ds(0, zc)],
                            acc.at[pl.ds(sid * RPT + t * zc, zc)])

        plsc.subcore_barrier()

        def gather(j, b):
            return pltpu.make_async_copy(x_hbm.at[src_v.at[j]],
                                         msgs_v[b], gsem[b])

        def scatter(j, b):
            return pltpu.make_async_copy(msgs_v[b],
                                         acc.at[dst_v.at[j]], ssem[b])

        for blk in range(NB):
            pltpu.sync_copy(src_hbm.at[wid, pl.ds(blk * CB, CB)], src_v)
            pltpu.sync_copy(dst_hbm.at[wid, pl.ds(blk * CB, CB)], dst_v)
            gather(0, 0).start()

            def pair(p, carry):
                for b in range(2):
                    j = 2 * p + b
                    gather(j, b).wait()
                    scatter(j, b).start(add=True)

                    @pl.when((j >= 1) & (j + 1 < CB))
                    def _():
                        scatter(j - 1, 1 - b).wait()

                    @pl.when(j + 1 < CB)
                    def _():
                        gather(j + 1, 1 - b).start()
                return carry
            lax.fori_loop(0, CB // 2, pair, 0)
            scatter(CB - 2, 0).wait()
            scatter(CB - 1, 1).wait()

        plsc.subcore_barrier()
        pltpu.sync_copy(acc.at[pl.ds(sid * RPT, RPT)],
                        out_hbm.at[core, pl.ds(sid * RPT, RPT)])

    return body(xp, src3, dst3)


def _combine_body(p_ref, o_ref):
    o_ref[...] = p_ref[0] + p_ref[1]


def _tc_combine(partial):
    rb = 1000
    return pl.pallas_call(
        _combine_body,
        grid=(N // rb,),
        in_specs=[pl.BlockSpec((NC, rb, C), lambda i: (0, i, 0))],
        out_specs=pl.BlockSpec((rb, C), lambda i: (i, 0)),
        out_shape=jax.ShapeDtypeStruct((N, C), jnp.float32),
    )(partial)


def kernel(x, edge_index):
    xp = jnp.transpose(x.reshape(C, N))
    src3 = edge_index[0].astype(jnp.int32).reshape(NW, CH, K)
    dst3 = edge_index[1].astype(jnp.int32).reshape(NW, CH, K)
    partial = _sc_scatter_sum(xp, src3, dst3)
    out_nc = _tc_combine(partial)
    return jnp.transpose(out_nc).reshape(1, C, N, 1)


# speedup vs baseline: 1.1120x; 1.0713x over previous
import functools

import jax
import jax.numpy as jnp
from jax import lax
from jax.experimental import pallas as pl
from jax.experimental.pallas import tpu as pltpu
from jax.experimental.pallas import tpu_sc as plsc

N = 10000
C = 128
E = 320000
NC = 2
NS = 16
NW = NC * NS
K = 125
CH = 80
CB = 40
NB = CH // CB
EPT = CH * K
N_PAD = 10240
RPT = N_PAD // NS
LANES = 16


def _sc_scatter_sum(xp, e4):
    mesh = plsc.VectorSubcoreMesh(core_axis_name="c", subcore_axis_name="s")

    @functools.partial(
        pl.kernel,
        out_type=jax.ShapeDtypeStruct((NC, N_PAD, C), jnp.float32),
        mesh=mesh,
        scratch_types=[
            pltpu.VMEM((CB, K), jnp.int32),
            pltpu.VMEM((CB, K), jnp.int32),
            [pltpu.VMEM((K, C), jnp.float32)] * 2,
            pltpu.VMEM_SHARED((N_PAD, C), jnp.float32),
            [pltpu.SemaphoreType.DMA] * 2,
            [pltpu.SemaphoreType.DMA] * 2,
        ],
    )
    def body(x_hbm, e_hbm, out_hbm, src_v, dst_v, msgs_v, acc,
             gsem, ssem):
        core = lax.axis_index("c")
        sid = lax.axis_index("s")
        wid = core * NS + sid

        def zrow(i, carry):
            for j in range(C // LANES):
                msgs_v[0][i, pl.ds(j * LANES, LANES)] = jnp.zeros(
                    (LANES,), jnp.float32)
            return carry
        lax.fori_loop(0, K, zrow, 0)
        zc = 40
        for t in range(RPT // zc):
            pltpu.sync_copy(msgs_v[0].at[pl.ds(0, zc)],
                            acc.at[pl.ds(sid * RPT + t * zc, zc)])

        plsc.subcore_barrier()

        def gather(j, b):
            return pltpu.make_async_copy(x_hbm.at[src_v.at[j]],
                                         msgs_v[b], gsem[b])

        def scatter(j, b):
            return pltpu.make_async_copy(msgs_v[b],
                                         acc.at[dst_v.at[j]], ssem[b])

        for blk in range(NB):
            pltpu.sync_copy(e_hbm.at[0, wid, pl.ds(blk * CB, CB)], src_v)
            pltpu.sync_copy(e_hbm.at[1, wid, pl.ds(blk * CB, CB)], dst_v)
            gather(0, 0).start()

            def pair(p, carry):
                for b in range(2):
                    j = 2 * p + b
                    gather(j, b).wait()
                    scatter(j, b).start(add=True)

                    @pl.when((j >= 1) & (j + 1 < CB))
                    def _():
                        scatter(j - 1, 1 - b).wait()

                    @pl.when(j + 1 < CB)
                    def _():
                        gather(j + 1, 1 - b).start()
                return carry
            lax.fori_loop(0, CB // 2, pair, 0)
            scatter(CB - 2, 0).wait()
            scatter(CB - 1, 1).wait()

        plsc.subcore_barrier()
        pltpu.sync_copy(acc.at[pl.ds(sid * RPT, RPT)],
                        out_hbm.at[core, pl.ds(sid * RPT, RPT)])

    return body(xp, e4)


def _combine_body(p_ref, o_ref):
    o_ref[...] = p_ref[0] + p_ref[1]


def _tc_combine(partial):
    rb = 1000
    return pl.pallas_call(
        _combine_body,
        grid=(N // rb,),
        in_specs=[pl.BlockSpec((NC, rb, C), lambda i: (0, i, 0))],
        out_specs=pl.BlockSpec((rb, C), lambda i: (i, 0)),
        out_shape=jax.ShapeDtypeStruct((N, C), jnp.float32),
    )(partial)


def kernel(x, edge_index):
    xp = jnp.transpose(x.reshape(C, N))
    e4 = edge_index.astype(jnp.int32).reshape(2, NW, CH, K)
    partial = _sc_scatter_sum(xp, e4)
    out_nc = _tc_combine(partial)
    return jnp.transpose(out_nc).reshape(1, C, N, 1)


# speedup vs baseline: 1.1253x; 1.0120x over previous
import functools

import jax
import jax.numpy as jnp
from jax import lax
from jax.experimental import pallas as pl
from jax.experimental.pallas import tpu as pltpu
from jax.experimental.pallas import tpu_sc as plsc

N = 10000
C = 128
E = 320000
NC = 2
NS = 16
NW = NC * NS
K = 125
CH = 80
CB = 40
NB = CH // CB
EPT = CH * K
N_PAD = 10240
RPT = N_PAD // NS
LANES = 16


def _sc_scatter_sum(xp, e4):
    mesh = plsc.VectorSubcoreMesh(core_axis_name="c", subcore_axis_name="s")

    @functools.partial(
        pl.kernel,
        out_type=jax.ShapeDtypeStruct((NC, N_PAD, C), jnp.float32),
        mesh=mesh,
        scratch_types=[
            pltpu.VMEM((CB, K), jnp.int32),
            pltpu.VMEM((CB, K), jnp.int32),
            [pltpu.VMEM((K, C), jnp.float32)] * 2,
            pltpu.VMEM_SHARED((N_PAD, C), jnp.float32),
            [pltpu.SemaphoreType.DMA] * 2,
            [pltpu.SemaphoreType.DMA] * 2,
        ],
    )
    def body(x_hbm, e_hbm, out_hbm, src_v, dst_v, msgs_v, acc,
             gsem, ssem):
        core = lax.axis_index("c")
        sid = lax.axis_index("s")
        wid = core * NS + sid

        def zrow(i, carry):
            for j in range(C // LANES):
                msgs_v[0][i, pl.ds(j * LANES, LANES)] = jnp.zeros(
                    (LANES,), jnp.float32)
            return carry
        lax.fori_loop(0, K, zrow, 0)
        zc = 40
        for t in range(RPT // zc):
            pltpu.sync_copy(msgs_v[0].at[pl.ds(0, zc)],
                            acc.at[pl.ds(sid * RPT + t * zc, zc)])

        plsc.subcore_barrier()

        def gather(j, b):
            return pltpu.make_async_copy(x_hbm.at[src_v.at[j]],
                                         msgs_v[b], gsem[b])

        def scatter(j, b):
            return pltpu.make_async_copy(msgs_v[b],
                                         acc.at[dst_v.at[j]], ssem[b])

        for blk in range(NB):
            pltpu.sync_copy(e_hbm.at[0, wid, pl.ds(blk * CB, CB)], src_v)
            pltpu.sync_copy(e_hbm.at[1, wid, pl.ds(blk * CB, CB)], dst_v)
            gather(0, 0).start()

            def pair(p, carry):
                for b in range(2):
                    j = 2 * p + b
                    gather(j, b).wait()
                    scatter(j, b).start(add=True)

                    @pl.when((j >= 1) & (j + 1 < CB))
                    def _():
                        scatter(j - 1, 1 - b).wait()

                    @pl.when(j + 1 < CB)
                    def _():
                        gather(j + 1, 1 - b).start()
                return carry
            lax.fori_loop(0, CB // 2, pair, 0)
            scatter(CB - 2, 0).wait()
            scatter(CB - 1, 1).wait()

        plsc.subcore_barrier()
        pltpu.sync_copy(acc.at[pl.ds(sid * RPT, RPT)],
                        out_hbm.at[core, pl.ds(sid * RPT, RPT)])

    return body(xp, e4)


def _combine_body(p_ref, o_ref):
    o_ref[...] = p_ref[0] + p_ref[1]


def _tc_combine(partial):
    rb = 2000
    return pl.pallas_call(
        _combine_body,
        grid=(N // rb,),
        in_specs=[pl.BlockSpec((NC, rb, C), lambda i: (0, i, 0))],
        out_specs=pl.BlockSpec((rb, C), lambda i: (i, 0)),
        out_shape=jax.ShapeDtypeStruct((N, C), jnp.float32),
    )(partial)


def kernel(x, edge_index):
    xp = jnp.transpose(x.reshape(C, N))
    e4 = edge_index.astype(jnp.int32).reshape(2, NW, CH, K)
    partial = _sc_scatter_sum(xp, e4)
    out_nc = _tc_combine(partial)
    return jnp.transpose(out_nc).reshape(1, C, N, 1)
